# baseline (speedup 1.0000x reference)
import functools

import jax
import jax.numpy as jnp
from jax import lax
from jax.experimental import pallas as pl
from jax.experimental.pallas import tpu as pltpu
from jax.experimental.pallas import tpu_sc as plsc

FEATURE_SIZE = 1000000
FIELD_SIZE = 26
BATCH = 4096
MAX_LEN = 20

NUM_WORKERS = 32
BPW = BATCH // NUM_WORKERS
CHUNK = BPW * MAX_LEN
PER_FIELD = BATCH * MAX_LEN
NGROUP = BPW // 16
PHASES = ((0, 7), (7, 14), (14, 20), (20, 26))
NSLOT = 7


def _sc_kernel(vals_hbm, idx_hbm, seq_hbm, table_hbm, out_hbm,
               idxb, vb, wb, seq_buf, out_buf, table_sh,
               sem_i, sem_g, sem_v):
    info = plsc.get_sparse_core_info()
    nc = info.num_cores
    sid = lax.axis_index("s")
    wid = sid * nc + lax.axis_index("c")
    b0 = wid * BPW
    col0 = wid * CHUNK

    lane = lax.iota(jnp.int32, 16)

    @pl.when(sid == 0)
    def _():
        pltpu.sync_copy(table_hbm, table_sh)

    plsc.subcore_barrier()

    pltpu.sync_copy(seq_hbm.at[pl.ds(b0, BPW), :], seq_buf)

    def fire_idx(p):
        lo, hi = PHASES[p]
        handles = []
        for j in range(hi - lo):
            src0 = (lo + j) * PER_FIELD + col0
            handles.append(pltpu.async_copy(
                idx_hbm.at[pl.ds(src0, CHUNK)],
                idxb.at[pl.ds(j * CHUNK, CHUNK)], sem_i))
        return handles

    def compute_fields(lo, hi):
        def field_body(f, c):
            j = f - lo
            fvec = jnp.broadcast_to(f, (16,)).astype(jnp.int32)

            def group_body(g, c2):
                acc = jnp.zeros((16,), jnp.float32)
                row = g * 16 + lane
                vrow = j * BPW + row
                wbase = j * CHUNK + (g * 16 + lane) * MAX_LEN
                for l in range(MAX_LEN):
                    w = plsc.load_gather(wb, [wbase + l])
                    v = plsc.load_gather(vb, [vrow, lvec[l]])
                    acc = acc + w * v
                sq = plsc.load_gather(seq_buf, [row, fvec]).astype(jnp.float32)
                plsc.store_scatter(out_buf, [row, fvec], acc / sq)
                return c2

            lax.fori_loop(0, NGROUP, group_body, 0)
            return c

        lvec = [jnp.broadcast_to(l, (16,)).astype(jnp.int32)
                for l in range(MAX_LEN)]
        lax.fori_loop(lo, hi, field_body, 0)

    ih = fire_idx(0)
    for p, (lo, hi) in enumerate(PHASES):
        gh, vh = [], []
        for j in range(hi - lo):
            ih[j].wait()
            gh.append(pltpu.async_copy(
                table_sh.at[idxb.at[pl.ds(j * CHUNK, CHUNK)]],
                wb.at[pl.ds(j * CHUNK, CHUNK)], sem_g))
            r0 = (lo + j) * BATCH + b0
            vh.append(pltpu.async_copy(
                vals_hbm.at[pl.ds(r0, BPW), :],
                vb.at[pl.ds(j * BPW, BPW), :], sem_v))
        for h in gh:
            h.wait()
        if p + 1 < len(PHASES):
            ih = fire_idx(p + 1)
        for h in vh:
            h.wait()
        compute_fields(lo, hi)

    pltpu.sync_copy(out_buf, out_hbm.at[pl.ds(b0, BPW), :])


@jax.jit
def _first_order(vals2d, idx_flat, seq2d, table_flat):
    mesh = plsc.VectorSubcoreMesh(core_axis_name="c", subcore_axis_name="s")
    run = functools.partial(
        pl.kernel,
        out_type=jax.ShapeDtypeStruct((BATCH, FIELD_SIZE), jnp.float32),
        mesh=mesh,
        compiler_params=pltpu.CompilerParams(
            needs_layout_passes=False, use_tc_tiling_on_sc=False),
        scratch_types=[
            pltpu.VMEM((NSLOT * CHUNK,), jnp.int32),
            pltpu.VMEM((NSLOT * BPW, MAX_LEN), jnp.float32),
            pltpu.VMEM((NSLOT * CHUNK,), jnp.float32),
            pltpu.VMEM((BPW, FIELD_SIZE), jnp.int32),
            pltpu.VMEM((BPW, FIELD_SIZE), jnp.float32),
            pltpu.VMEM_SHARED((FEATURE_SIZE + 2,), jnp.float32),
            pltpu.SemaphoreType.DMA,
            pltpu.SemaphoreType.DMA,
            pltpu.SemaphoreType.DMA,
        ],
    )(_sc_kernel)
    return run(vals2d, idx_flat, seq2d, table_flat)


def kernel(feature_values, feature_idx, seq_lens, weights_first_order):
    idx_flat = feature_idx.astype(jnp.int32).reshape(FIELD_SIZE * PER_FIELD)
    table_flat = weights_first_order.reshape(FEATURE_SIZE + 2)
    return _first_order(feature_values, idx_flat, seq_lens, table_flat)


# speedup vs baseline: 2.0925x; 2.0925x over previous
import functools

import jax
import jax.numpy as jnp
from jax import lax
from jax.experimental import pallas as pl
from jax.experimental.pallas import tpu as pltpu
from jax.experimental.pallas import tpu_sc as plsc

FEATURE_SIZE = 1000000
FIELD_SIZE = 26
BATCH = 4096
MAX_LEN = 20

NUM_WORKERS = 32
NROWS = FIELD_SIZE * BATCH
RPW = NROWS // NUM_WORKERS
NPIECE = 8
PIECE = RPW // NPIECE
NGROUP = PIECE // 16


def _sc_kernel(vals_hbm, idx_hbm, seq_hbm, table_hbm, out_hbm,
               idxb0, idxb1, vb0, vb1, wb0, wb1, seq_buf, out_buf, table_sh,
               sem_i0, sem_i1, sem_g0, sem_g1, sem_v0, sem_v1):
    info = plsc.get_sparse_core_info()
    nc = info.num_cores
    sid = lax.axis_index("s")
    wid = sid * nc + lax.axis_index("c")
    r0 = wid * RPW

    idxb = (idxb0, idxb1)
    vb = (vb0, vb1)
    wb = (wb0, wb1)
    sem_i = (sem_i0, sem_i1)
    sem_g = (sem_g0, sem_g1)
    sem_v = (sem_v0, sem_v1)

    @pl.when(sid == 0)
    def _():
        pltpu.sync_copy(table_hbm, table_sh)

    plsc.subcore_barrier()

    pltpu.sync_copy(seq_hbm.at[pl.ds(r0, RPW)], seq_buf)

    def fire_idx(p):
        s = p & 1
        return pltpu.async_copy(
            idx_hbm.at[:, pl.ds(r0 + p * PIECE, PIECE)], idxb[s], sem_i[s])

    def fire_vals(p):
        s = p & 1
        return pltpu.async_copy(
            vals_hbm.at[:, pl.ds(r0 + p * PIECE, PIECE)], vb[s], sem_v[s])

    def fire_gathers(p):
        s = p & 1
        return [pltpu.async_copy(table_sh.at[idxb[s].at[l]], wb[s].at[l],
                                 sem_g[s])
                for l in range(MAX_LEN)]

    def compute(p):
        s = p & 1
        w_r, v_r = wb[s], vb[s]

        def group_body(g, c):
            sl = pl.ds(g * 16, 16)
            acc = w_r[0, sl] * v_r[0, sl]
            for l in range(1, MAX_LEN):
                acc = acc + w_r[l, sl] * v_r[l, sl]
            osl = pl.ds(p * PIECE + g * 16, 16)
            out_buf[osl] = acc / seq_buf[osl].astype(jnp.float32)
            return c

        lax.fori_loop(0, NGROUP, group_body, 0)

    ih = {0: fire_idx(0), 1: fire_idx(1)}
    ih[0].wait()
    pend = {0: (fire_gathers(0), fire_vals(0))}
    for p in range(NPIECE):
        if p + 1 < NPIECE:
            ih[p + 1].wait()
            pend[p + 1] = (fire_gathers(p + 1), fire_vals(p + 1))
        gh, vh = pend[p]
        for h in gh:
            h.wait()
        vh.wait()
        if p + 2 < NPIECE:
            ih[p + 2] = fire_idx(p + 2)
        compute(p)

    pltpu.sync_copy(out_buf, out_hbm.at[pl.ds(r0, RPW)])


@jax.jit
def _first_order(vals_t, idx_t, seq_r, table_flat):
    mesh = plsc.VectorSubcoreMesh(core_axis_name="c", subcore_axis_name="s")
    run = functools.partial(
        pl.kernel,
        out_type=jax.ShapeDtypeStruct((NROWS,), jnp.float32),
        mesh=mesh,
        compiler_params=pltpu.CompilerParams(
            needs_layout_passes=False, use_tc_tiling_on_sc=False),
        scratch_types=[
            pltpu.VMEM((MAX_LEN, PIECE), jnp.int32),
            pltpu.VMEM((MAX_LEN, PIECE), jnp.int32),
            pltpu.VMEM((MAX_LEN, PIECE), jnp.float32),
            pltpu.VMEM((MAX_LEN, PIECE), jnp.float32),
            pltpu.VMEM((MAX_LEN, PIECE), jnp.float32),
            pltpu.VMEM((MAX_LEN, PIECE), jnp.float32),
            pltpu.VMEM((RPW,), jnp.int32),
            pltpu.VMEM((RPW,), jnp.float32),
            pltpu.VMEM_SHARED((FEATURE_SIZE + 2,), jnp.float32),
            pltpu.SemaphoreType.DMA,
            pltpu.SemaphoreType.DMA,
            pltpu.SemaphoreType.DMA,
            pltpu.SemaphoreType.DMA,
            pltpu.SemaphoreType.DMA,
            pltpu.SemaphoreType.DMA,
        ],
    )(_sc_kernel)
    return run(vals_t, idx_t, seq_r, table_flat)


def kernel(feature_values, feature_idx, seq_lens, weights_first_order):
    vals_t = feature_values.T
    idx_t = feature_idx.astype(jnp.int32).T
    seq_r = seq_lens.T.reshape(NROWS)
    table_flat = weights_first_order.T.reshape(FEATURE_SIZE + 2)
    out_r = _first_order(vals_t, idx_t, seq_r, table_flat)
    return out_r.reshape(FIELD_SIZE, BATCH).T
